# baseline (speedup 1.0000x reference)
import functools

import jax
import jax.numpy as jnp
from jax import lax
from jax.experimental import pallas as pl
from jax.experimental.pallas import tpu as pltpu
from jax.experimental.pallas import tpu_sc as plsc


def _bow_kernel(B, H, V, D):
    info = plsc.get_sparse_core_info()
    NC, NS, L = info.num_cores, info.num_subcores, info.num_lanes
    NW = NC * NS
    assert B % NW == 0 and D % L == 0
    bpw = B // NW

    mesh = plsc.VectorSubcoreMesh(core_axis_name="c", subcore_axis_name="s")

    @functools.partial(
        pl.kernel,
        mesh=mesh,
        out_type=jax.ShapeDtypeStruct((B, D), jnp.float32),
        scratch_types=[
            pltpu.VMEM((bpw, H), jnp.int32),
            pltpu.VMEM((2, H, D), jnp.float32),
            pltpu.VMEM((bpw, D), jnp.float32),
            pltpu.SemaphoreType.DMA,
            pltpu.SemaphoreType.DMA,
        ],
        compiler_params=pltpu.CompilerParams(use_tc_tiling_on_sc=False),
    )
    def body(x_hbm, tab_hbm, out_hbm, idx_v, rows_v, out_v, sem0, sem1):
        wid = lax.axis_index("s") * NC + lax.axis_index("c")
        base = wid * bpw
        pltpu.sync_copy(x_hbm.at[pl.ds(base, bpw), :], idx_v)

        sems = (sem0, sem1)
        inv_h = jnp.float32(1.0 / H)

        for k in range(2):
            pltpu.async_copy(tab_hbm.at[idx_v.at[k]], rows_v.at[k], sems[k])

        def step(b2, carry):
            for k in range(2):
                b = b2 * 2 + k
                pltpu.make_async_copy(
                    tab_hbm.at[idx_v.at[0]], rows_v.at[k], sems[k]
                ).wait()
                rows = rows_v.at[k]
                for d in range(D // L):
                    sl = pl.ds(d * L, L)
                    acc = rows[0, sl]
                    for j in range(1, H):
                        acc = acc + rows[j, sl]
                    out_v[b, sl] = acc * inv_h
                nb = jnp.minimum(b + 2, bpw - 1)
                pltpu.async_copy(tab_hbm.at[idx_v.at[nb]], rows_v.at[k], sems[k])

            return carry

        lax.fori_loop(0, bpw // 2, step, 0)

        for k in range(2):
            pltpu.make_async_copy(
                tab_hbm.at[idx_v.at[0]], rows_v.at[k], sems[k]
            ).wait()

        pltpu.sync_copy(out_v, out_hbm.at[pl.ds(base, bpw), :])

    return body


def kernel(x, table):
    B, H = x.shape
    V, D = table.shape
    x = x.astype(jnp.int32)
    return _bow_kernel(B, H, V, D)(x, table)
